# baseline (speedup 1.0000x reference)
import functools

import jax
import jax.numpy as jnp
from jax.experimental import pallas as pl

HID = 1024
NH = 16
HD = HID // NH
FFN = 2816
PFFN = 512
EPS = 1e-6
BETA_CE = 1.0
BETA_CU = 1.0
CAP = 0.5

F32 = jnp.float32
BF16 = jnp.bfloat16


def _rms(x, w):
    v = jnp.mean(x * x, axis=-1, keepdims=True)
    return x * jax.lax.rsqrt(v + EPS) * w


def _qkv_kernel(x_ref, ln_ref, wq_ref, wk_ref, wv_ref, cos_ref, sin_ref,
                q_ref, k_ref, v_ref):
    h = _rms(x_ref[0], ln_ref[...]).astype(BF16)
    q = jnp.dot(h, wq_ref[...], preferred_element_type=F32)
    k = jnp.dot(h, wk_ref[...], preferred_element_type=F32)
    v = jnp.dot(h, wv_ref[...], preferred_element_type=F32)
    cos = cos_ref[...]
    sin = sin_ref[...]
    half = HD // 2
    for n in range(NH):
        sl = slice(n * HD, (n + 1) * HD)
        qn = q[:, sl]
        kn = k[:, sl]
        qrot = jnp.concatenate([-qn[:, half:], qn[:, :half]], axis=1)
        krot = jnp.concatenate([-kn[:, half:], kn[:, :half]], axis=1)
        q_ref[0, n] = (qn * cos + qrot * sin).astype(BF16)
        k_ref[0, n] = (kn * cos + krot * sin).astype(BF16)
        v_ref[0, n] = v[:, sl].astype(BF16)


def _attn_kernel(q_ref, k_ref, v_ref, o_ref):
    q = q_ref[0, 0]
    k = k_ref[0, 0]
    v = v_ref[0, 0]
    s = jax.lax.dot_general(q, k, (((1,), (1,)), ((), ())),
                            preferred_element_type=F32) * (1.0 / 8.0)
    base = pl.program_id(2) * q.shape[0]
    row = jax.lax.broadcasted_iota(jnp.int32, s.shape, 0) + base
    col = jax.lax.broadcasted_iota(jnp.int32, s.shape, 1)
    s = jnp.where(col <= row, s, -1e9)
    m = jnp.max(s, axis=1, keepdims=True)
    p = jnp.exp(s - m)
    o = jax.lax.dot_general(p.astype(BF16), v, (((1,), (0,)), ((), ())),
                            preferred_element_type=F32)
    o_ref[0, 0] = (o / jnp.sum(p, axis=1, keepdims=True)).astype(BF16)


def _oproj_kernel(o_ref, wo_ref, x_ref, y_ref):
    oc = jnp.concatenate([o_ref[0, n] for n in range(NH)], axis=1)
    y_ref[0] = x_ref[0] + jnp.dot(oc, wo_ref[...], preferred_element_type=F32)


def _mlp_kernel(x_ref, ln_ref, wg_ref, wu_ref, wd_ref, o_ref):
    x = x_ref[0]
    h = _rms(x, ln_ref[...]).astype(BF16)
    g = jnp.dot(h, wg_ref[...], preferred_element_type=F32)
    u = jnp.dot(h, wu_ref[...], preferred_element_type=F32)
    a = (g * jax.nn.sigmoid(g) * u).astype(BF16)
    part = jnp.dot(a, wd_ref[...], preferred_element_type=F32)

    @pl.when(pl.program_id(2) == 0)
    def _init():
        o_ref[0] = x + part

    @pl.when(pl.program_id(2) > 0)
    def _acc():
        o_ref[0] += part


def _gate_kernel(x_ref, proc_ref, prior_ref, g_ref, dch_ref):
    x = x_ref[0]
    p = proc_ref[0]
    q = prior_ref[0]
    ar = p - x
    dst = jnp.sum(ar * ar, axis=1, keepdims=True) * (1.0 / HID)
    df = p - q
    dch = jnp.sum(df * df, axis=1, keepdims=True) * (1.0 / HID)
    g_ref[0] = jax.nn.sigmoid(BETA_CE * dst - BETA_CU * dch)
    s = jnp.reshape(jnp.sum(dch), (1, 1, 1))

    @pl.when(pl.program_id(1) == 0)
    def _init():
        dch_ref[...] = s

    @pl.when(pl.program_id(1) > 0)
    def _acc():
        dch_ref[...] += s


def _rank_kernel(gc_ref, gr_ref, rank_ref):
    gc = gc_ref[0]
    gr = gr_ref[0]
    bs = gc.shape[0]
    s = gr.shape[1]
    base = pl.program_id(1) * bs
    i = jax.lax.broadcasted_iota(jnp.int32, (bs, s), 0) + base
    j = jax.lax.broadcasted_iota(jnp.int32, (bs, s), 1)
    gt = (gr > gc).astype(F32)
    eq = ((gr == gc) & (j < i)).astype(F32)
    rank_ref[0] = jnp.sum(gt + eq, axis=1, keepdims=True)


def _gather_kernel(rank_ref, g_ref, proc_ref, sel_ref, gate_ref, *, K):
    r = rank_ref[0].astype(jnp.int32)
    jr = jax.lax.broadcasted_iota(jnp.int32, (r.shape[0], K), 1)
    q = (r == jr).astype(F32)
    selp = jax.lax.dot_general(q, proc_ref[0], (((0,), (0,)), ((), ())),
                               preferred_element_type=F32)
    gatep = jax.lax.dot_general(q, g_ref[0], (((0,), (0,)), ((), ())),
                                preferred_element_type=F32)

    @pl.when(pl.program_id(1) == 0)
    def _init():
        sel_ref[0] = selp
        gate_ref[0] = gatep

    @pl.when(pl.program_id(1) > 0)
    def _acc():
        sel_ref[0] += selp
        gate_ref[0] += gatep


def _scatter_kernel(rank_ref, proc_ref, sel_ref, dyn_ref, gate_ref, fin_ref,
                    *, K):
    r = rank_ref[0].astype(jnp.int32)
    w = (r == jax.lax.broadcasted_iota(jnp.int32,
                                       (r.shape[0], K), 1)).astype(F32)
    sel = sel_ref[0]
    dyn = dyn_ref[0]
    gate = gate_ref[0]
    gated = sel + gate * (dyn - sel)
    msel = (r < K).astype(F32)
    fin_ref[0] = (jnp.dot(w, gated, preferred_element_type=F32)
                  + (1.0 - msel) * proc_ref[0])


def _rope_tables(s):
    pos = jnp.arange(s, dtype=F32)
    inv = 1.0 / (10000.0 ** (jnp.arange(0, HD, 2, dtype=F32) / HD))
    f = pos[:, None] * inv[None, :]
    emb = jnp.concatenate([f, f], axis=-1)
    return jnp.cos(emb), jnp.sin(emb)


def _decoder_layer(x, p, pre, cos, sin):
    b, s, _ = x.shape
    bs = min(256, s)
    bq = min(512, s)
    nb = s // bs
    ln1 = p[pre + "_ln1"].reshape(1, HID)

    q, k, v = pl.pallas_call(
        _qkv_kernel,
        grid=(b, nb),
        in_specs=[
            pl.BlockSpec((1, bs, HID), lambda bi, si: (bi, si, 0)),
            pl.BlockSpec((1, HID), lambda bi, si: (0, 0)),
            pl.BlockSpec((HID, HID), lambda bi, si: (0, 0)),
            pl.BlockSpec((HID, HID), lambda bi, si: (0, 0)),
            pl.BlockSpec((HID, HID), lambda bi, si: (0, 0)),
            pl.BlockSpec((bs, HD), lambda bi, si: (si, 0)),
            pl.BlockSpec((bs, HD), lambda bi, si: (si, 0)),
        ],
        out_specs=[
            pl.BlockSpec((1, NH, bs, HD), lambda bi, si: (bi, 0, si, 0)),
            pl.BlockSpec((1, NH, bs, HD), lambda bi, si: (bi, 0, si, 0)),
            pl.BlockSpec((1, NH, bs, HD), lambda bi, si: (bi, 0, si, 0)),
        ],
        out_shape=[jax.ShapeDtypeStruct((b, NH, s, HD), BF16)] * 3,
    )(x, ln1, p[pre + "_wq"].astype(BF16), p[pre + "_wk"].astype(BF16),
      p[pre + "_wv"].astype(BF16), cos, sin)

    o = pl.pallas_call(
        _attn_kernel,
        grid=(b, NH, s // bq),
        in_specs=[
            pl.BlockSpec((1, 1, bq, HD), lambda bi, hi, si: (bi, hi, si, 0)),
            pl.BlockSpec((1, 1, s, HD), lambda bi, hi, si: (bi, hi, 0, 0)),
            pl.BlockSpec((1, 1, s, HD), lambda bi, hi, si: (bi, hi, 0, 0)),
        ],
        out_specs=pl.BlockSpec((1, 1, bq, HD),
                               lambda bi, hi, si: (bi, hi, si, 0)),
        out_shape=jax.ShapeDtypeStruct((b, NH, s, HD), BF16),
    )(q, k, v)

    x2 = pl.pallas_call(
        _oproj_kernel,
        grid=(b, nb),
        in_specs=[
            pl.BlockSpec((1, NH, bs, HD), lambda bi, si: (bi, 0, si, 0)),
            pl.BlockSpec((HID, HID), lambda bi, si: (0, 0)),
            pl.BlockSpec((1, bs, HID), lambda bi, si: (bi, si, 0)),
        ],
        out_specs=pl.BlockSpec((1, bs, HID), lambda bi, si: (bi, si, 0)),
        out_shape=jax.ShapeDtypeStruct(x.shape, F32),
    )(o, p[pre + "_wo"].astype(BF16), x)

    return _mlp(x2, p[pre + "_ln2"], p[pre + "_wg"], p[pre + "_wu"],
                p[pre + "_wd"])


def _mlp(x, ln, wg, wu, wd, fb=None):
    b, s, _ = x.shape
    bs = min(256, s)
    nb = s // bs
    f = wg.shape[1]
    if fb is None:
        fb = f
    nf = f // fb
    return pl.pallas_call(
        _mlp_kernel,
        grid=(b, nb, nf),
        in_specs=[
            pl.BlockSpec((1, bs, HID), lambda bi, si, fi: (bi, si, 0)),
            pl.BlockSpec((1, HID), lambda bi, si, fi: (0, 0)),
            pl.BlockSpec((HID, fb), lambda bi, si, fi: (0, fi)),
            pl.BlockSpec((HID, fb), lambda bi, si, fi: (0, fi)),
            pl.BlockSpec((fb, HID), lambda bi, si, fi: (fi, 0)),
        ],
        out_specs=pl.BlockSpec((1, bs, HID), lambda bi, si, fi: (bi, si, 0)),
        out_shape=jax.ShapeDtypeStruct(x.shape, F32),
    )(x, ln.reshape(1, HID), wg.astype(BF16), wu.astype(BF16),
      wd.astype(BF16))


def kernel(hidden_states, params):
    x = hidden_states
    b, s, _ = x.shape
    k = max(1, int(s * CAP))
    bs = min(256, s)
    nb = s // bs

    cos_s, sin_s = _rope_tables(s)
    processed = _decoder_layer(x, params, "dec", cos_s, sin_s)
    prior_hidden = _mlp(x, params["prior_ln"], params["prior_wg"],
                        params["prior_wu"], params["prior_wd"])

    g, dchsum = pl.pallas_call(
        _gate_kernel,
        grid=(b, nb),
        in_specs=[
            pl.BlockSpec((1, bs, HID), lambda bi, si: (bi, si, 0)),
            pl.BlockSpec((1, bs, HID), lambda bi, si: (bi, si, 0)),
            pl.BlockSpec((1, bs, HID), lambda bi, si: (bi, si, 0)),
        ],
        out_specs=[
            pl.BlockSpec((1, bs, 1), lambda bi, si: (bi, si, 0)),
            pl.BlockSpec((1, 1, 1), lambda bi, si: (bi, 0, 0)),
        ],
        out_shape=[
            jax.ShapeDtypeStruct((b, s, 1), F32),
            jax.ShapeDtypeStruct((b, 1, 1), F32),
        ],
    )(x, processed, prior_hidden)
    prior_loss = jnp.sum(dchsum) / (b * s)

    gr = g.reshape(b, 1, s)
    rank = pl.pallas_call(
        _rank_kernel,
        grid=(b, nb),
        in_specs=[
            pl.BlockSpec((1, bs, 1), lambda bi, si: (bi, si, 0)),
            pl.BlockSpec((1, 1, s), lambda bi, si: (bi, 0, 0)),
        ],
        out_specs=pl.BlockSpec((1, bs, 1), lambda bi, si: (bi, si, 0)),
        out_shape=jax.ShapeDtypeStruct((b, s, 1), F32),
    )(g, gr)

    sel, gate = pl.pallas_call(
        functools.partial(_gather_kernel, K=k),
        grid=(b, nb),
        in_specs=[
            pl.BlockSpec((1, bs, 1), lambda bi, si: (bi, si, 0)),
            pl.BlockSpec((1, bs, 1), lambda bi, si: (bi, si, 0)),
            pl.BlockSpec((1, bs, HID), lambda bi, si: (bi, si, 0)),
        ],
        out_specs=[
            pl.BlockSpec((1, k, HID), lambda bi, si: (bi, 0, 0)),
            pl.BlockSpec((1, k, 1), lambda bi, si: (bi, 0, 0)),
        ],
        out_shape=[
            jax.ShapeDtypeStruct((b, k, HID), F32),
            jax.ShapeDtypeStruct((b, k, 1), F32),
        ],
    )(rank, g, processed)

    cos_k, sin_k = _rope_tables(k)
    dyn = _decoder_layer(sel, params, "dyn", cos_k, sin_k)

    fin = pl.pallas_call(
        functools.partial(_scatter_kernel, K=k),
        grid=(b, nb),
        in_specs=[
            pl.BlockSpec((1, bs, 1), lambda bi, si: (bi, si, 0)),
            pl.BlockSpec((1, bs, HID), lambda bi, si: (bi, si, 0)),
            pl.BlockSpec((1, k, HID), lambda bi, si: (bi, 0, 0)),
            pl.BlockSpec((1, k, HID), lambda bi, si: (bi, 0, 0)),
            pl.BlockSpec((1, k, 1), lambda bi, si: (bi, 0, 0)),
        ],
        out_specs=pl.BlockSpec((1, bs, HID), lambda bi, si: (bi, si, 0)),
        out_shape=jax.ShapeDtypeStruct((b, s, HID), F32),
    )(rank, processed, sel, dyn, gate)

    return (fin, prior_loss)


# speedup vs baseline: 1.3682x; 1.3682x over previous
import functools

import jax
import jax.numpy as jnp
from jax.experimental import pallas as pl

HID = 1024
NH = 16
HD = HID // NH
FFN = 2816
PFFN = 512
EPS = 1e-6
BETA_CE = 1.0
BETA_CU = 1.0
CAP = 0.5

F32 = jnp.float32
BF16 = jnp.bfloat16


def _rms(x, w):
    v = jnp.mean(x * x, axis=-1, keepdims=True)
    return x * jax.lax.rsqrt(v + EPS) * w


def _qkv_kernel(x_ref, ln_ref, wq_ref, wk_ref, wv_ref, cos_ref, sin_ref,
                q_ref, k_ref, v_ref):
    h = _rms(x_ref[0], ln_ref[...])
    q = jnp.dot(h, wq_ref[...], preferred_element_type=F32)
    k = jnp.dot(h, wk_ref[...], preferred_element_type=F32)
    v = jnp.dot(h, wv_ref[...], preferred_element_type=F32)
    cos = cos_ref[...]
    sin = sin_ref[...]
    half = HD // 2
    for n in range(NH):
        sl = slice(n * HD, (n + 1) * HD)
        qn = q[:, sl]
        kn = k[:, sl]
        qrot = jnp.concatenate([-qn[:, half:], qn[:, :half]], axis=1)
        krot = jnp.concatenate([-kn[:, half:], kn[:, :half]], axis=1)
        q_ref[0, n] = qn * cos + qrot * sin
        k_ref[0, n] = kn * cos + krot * sin
        v_ref[0, n] = v[:, sl]


def _attn_kernel(q_ref, k_ref, v_ref, o_ref):
    q = q_ref[0, 0]
    si = pl.program_id(2)
    bq = q.shape[0]
    hd = q.shape[1]

    def body(j, carry):
        l, acc = carry
        k = k_ref[0, 0, pl.ds(j * bq, bq), :]
        v = v_ref[0, 0, pl.ds(j * bq, bq), :]
        s = jax.lax.dot_general(q, k, (((1,), (1,)), ((), ())),
                                preferred_element_type=F32) * 0.125
        p = jnp.exp(s)
        l = l + jnp.sum(p, axis=1, keepdims=True)
        acc = acc + jax.lax.dot_general(p, v, (((1,), (0,)), ((), ())),
                                        preferred_element_type=F32)
        return l, acc

    init = (jnp.zeros((bq, 1), F32), jnp.zeros((bq, hd), F32))
    l, acc = jax.lax.fori_loop(0, si, body, init)

    k = k_ref[0, 0, pl.ds(si * bq, bq), :]
    v = v_ref[0, 0, pl.ds(si * bq, bq), :]
    s = jax.lax.dot_general(q, k, (((1,), (1,)), ((), ())),
                            preferred_element_type=F32) * 0.125
    row = jax.lax.broadcasted_iota(jnp.int32, (bq, bq), 0)
    col = jax.lax.broadcasted_iota(jnp.int32, (bq, bq), 1)
    p = jnp.where(col <= row, jnp.exp(s), 0.0)
    l = l + jnp.sum(p, axis=1, keepdims=True)
    acc = acc + jax.lax.dot_general(p, v, (((1,), (0,)), ((), ())),
                                    preferred_element_type=F32)
    o_ref[0, 0] = acc / l


def _oproj_kernel(o_ref, wo_ref, x_ref, y_ref):
    oc = jnp.concatenate([o_ref[0, n] for n in range(NH)], axis=1)
    y_ref[0] = x_ref[0] + jnp.dot(oc, wo_ref[...], preferred_element_type=F32)


def _mlp_kernel(x_ref, ln_ref, wg_ref, wu_ref, wd_ref, o_ref):
    x = x_ref[0]
    h = _rms(x, ln_ref[...])
    g = jnp.dot(h, wg_ref[...], preferred_element_type=F32)
    u = jnp.dot(h, wu_ref[...], preferred_element_type=F32)
    a = g * jax.nn.sigmoid(g) * u
    part = jnp.dot(a, wd_ref[...], preferred_element_type=F32)

    @pl.when(pl.program_id(2) == 0)
    def _init():
        o_ref[0] = x + part

    @pl.when(pl.program_id(2) > 0)
    def _acc():
        o_ref[0] += part


def _gate_kernel(x_ref, proc_ref, prior_ref, g_ref, dch_ref):
    x = x_ref[0]
    p = proc_ref[0]
    q = prior_ref[0]
    ar = p - x
    dst = jnp.sum(ar * ar, axis=1, keepdims=True) * (1.0 / HID)
    df = p - q
    dch = jnp.sum(df * df, axis=1, keepdims=True) * (1.0 / HID)
    g_ref[0] = jax.nn.sigmoid(BETA_CE * dst - BETA_CU * dch)
    s = jnp.reshape(jnp.sum(dch), (1, 1, 1))

    @pl.when(pl.program_id(1) == 0)
    def _init():
        dch_ref[...] = s

    @pl.when(pl.program_id(1) > 0)
    def _acc():
        dch_ref[...] += s


def _rank_kernel(gc_ref, gr_ref, rank_ref):
    gc = gc_ref[0]
    gr = gr_ref[0]
    bs = gc.shape[0]
    s = gr.shape[1]
    base = pl.program_id(1) * bs
    i = jax.lax.broadcasted_iota(jnp.int32, (bs, s), 0) + base
    j = jax.lax.broadcasted_iota(jnp.int32, (bs, s), 1)
    gt = (gr > gc).astype(F32)
    eq = ((gr == gc) & (j < i)).astype(F32)
    rank_ref[0] = jnp.sum(gt + eq, axis=1, keepdims=True)


def _gather_kernel(rank_ref, g_ref, proc_ref, sel_ref, gate_ref, *, K):
    r = rank_ref[0].astype(jnp.int32)
    jr = jax.lax.broadcasted_iota(jnp.int32, (r.shape[0], K), 1)
    q = (r == jr).astype(F32)
    selp = jax.lax.dot_general(q, proc_ref[0], (((0,), (0,)), ((), ())),
                               preferred_element_type=F32)
    gatep = jax.lax.dot_general(q, g_ref[0], (((0,), (0,)), ((), ())),
                                preferred_element_type=F32)

    @pl.when(pl.program_id(1) == 0)
    def _init():
        sel_ref[0] = selp
        gate_ref[0] = gatep

    @pl.when(pl.program_id(1) > 0)
    def _acc():
        sel_ref[0] += selp
        gate_ref[0] += gatep


def _scatter_kernel(rank_ref, proc_ref, sel_ref, dyn_ref, gate_ref, fin_ref,
                    *, K):
    r = rank_ref[0].astype(jnp.int32)
    w = (r == jax.lax.broadcasted_iota(jnp.int32,
                                       (r.shape[0], K), 1)).astype(F32)
    sel = sel_ref[0]
    dyn = dyn_ref[0]
    gate = gate_ref[0]
    gated = sel + gate * (dyn - sel)
    msel = (r < K).astype(F32)
    fin_ref[0] = (jnp.dot(w, gated, preferred_element_type=F32)
                  + (1.0 - msel) * proc_ref[0])


def _rope_tables(s):
    pos = jnp.arange(s, dtype=F32)
    inv = 1.0 / (10000.0 ** (jnp.arange(0, HD, 2, dtype=F32) / HD))
    f = pos[:, None] * inv[None, :]
    emb = jnp.concatenate([f, f], axis=-1)
    return jnp.cos(emb), jnp.sin(emb)


def _decoder_layer(x, p, pre, cos, sin):
    b, s, _ = x.shape
    bs = min(256, s)
    bq = min(512, s)
    nb = s // bs
    ln1 = p[pre + "_ln1"].reshape(1, HID)

    q, k, v = pl.pallas_call(
        _qkv_kernel,
        grid=(b, nb),
        in_specs=[
            pl.BlockSpec((1, bs, HID), lambda bi, si: (bi, si, 0)),
            pl.BlockSpec((1, HID), lambda bi, si: (0, 0)),
            pl.BlockSpec((HID, HID), lambda bi, si: (0, 0)),
            pl.BlockSpec((HID, HID), lambda bi, si: (0, 0)),
            pl.BlockSpec((HID, HID), lambda bi, si: (0, 0)),
            pl.BlockSpec((bs, HD), lambda bi, si: (si, 0)),
            pl.BlockSpec((bs, HD), lambda bi, si: (si, 0)),
        ],
        out_specs=[
            pl.BlockSpec((1, NH, bs, HD), lambda bi, si: (bi, 0, si, 0)),
            pl.BlockSpec((1, NH, bs, HD), lambda bi, si: (bi, 0, si, 0)),
            pl.BlockSpec((1, NH, bs, HD), lambda bi, si: (bi, 0, si, 0)),
        ],
        out_shape=[jax.ShapeDtypeStruct((b, NH, s, HD), F32)] * 3,
    )(x, ln1, p[pre + "_wq"], p[pre + "_wk"], p[pre + "_wv"], cos, sin)

    o = pl.pallas_call(
        _attn_kernel,
        grid=(b, NH, s // bq),
        in_specs=[
            pl.BlockSpec((1, 1, bq, HD), lambda bi, hi, si: (bi, hi, si, 0)),
            pl.BlockSpec((1, 1, s, HD), lambda bi, hi, si: (bi, hi, 0, 0)),
            pl.BlockSpec((1, 1, s, HD), lambda bi, hi, si: (bi, hi, 0, 0)),
        ],
        out_specs=pl.BlockSpec((1, 1, bq, HD),
                               lambda bi, hi, si: (bi, hi, si, 0)),
        out_shape=jax.ShapeDtypeStruct((b, NH, s, HD), F32),
    )(q, k, v)

    x2 = pl.pallas_call(
        _oproj_kernel,
        grid=(b, nb),
        in_specs=[
            pl.BlockSpec((1, NH, bs, HD), lambda bi, si: (bi, 0, si, 0)),
            pl.BlockSpec((HID, HID), lambda bi, si: (0, 0)),
            pl.BlockSpec((1, bs, HID), lambda bi, si: (bi, si, 0)),
        ],
        out_specs=pl.BlockSpec((1, bs, HID), lambda bi, si: (bi, si, 0)),
        out_shape=jax.ShapeDtypeStruct(x.shape, F32),
    )(o, p[pre + "_wo"], x)

    return _mlp(x2, p[pre + "_ln2"], p[pre + "_wg"], p[pre + "_wu"],
                p[pre + "_wd"])


def _mlp(x, ln, wg, wu, wd, fb=None):
    b, s, _ = x.shape
    bs = min(256, s)
    nb = s // bs
    f = wg.shape[1]
    if fb is None:
        fb = f
    nf = f // fb
    return pl.pallas_call(
        _mlp_kernel,
        grid=(b, nb, nf),
        in_specs=[
            pl.BlockSpec((1, bs, HID), lambda bi, si, fi: (bi, si, 0)),
            pl.BlockSpec((1, HID), lambda bi, si, fi: (0, 0)),
            pl.BlockSpec((HID, fb), lambda bi, si, fi: (0, fi)),
            pl.BlockSpec((HID, fb), lambda bi, si, fi: (0, fi)),
            pl.BlockSpec((fb, HID), lambda bi, si, fi: (fi, 0)),
        ],
        out_specs=pl.BlockSpec((1, bs, HID), lambda bi, si, fi: (bi, si, 0)),
        out_shape=jax.ShapeDtypeStruct(x.shape, F32),
    )(x, ln.reshape(1, HID), wg, wu, wd)


def kernel(hidden_states, params):
    x = hidden_states
    b, s, _ = x.shape
    k = max(1, int(s * CAP))
    bs = min(256, s)
    nb = s // bs

    cos_s, sin_s = _rope_tables(s)
    processed = _decoder_layer(x, params, "dec", cos_s, sin_s)
    prior_hidden = _mlp(x, params["prior_ln"], params["prior_wg"],
                        params["prior_wu"], params["prior_wd"])

    g, dchsum = pl.pallas_call(
        _gate_kernel,
        grid=(b, nb),
        in_specs=[
            pl.BlockSpec((1, bs, HID), lambda bi, si: (bi, si, 0)),
            pl.BlockSpec((1, bs, HID), lambda bi, si: (bi, si, 0)),
            pl.BlockSpec((1, bs, HID), lambda bi, si: (bi, si, 0)),
        ],
        out_specs=[
            pl.BlockSpec((1, bs, 1), lambda bi, si: (bi, si, 0)),
            pl.BlockSpec((1, 1, 1), lambda bi, si: (bi, 0, 0)),
        ],
        out_shape=[
            jax.ShapeDtypeStruct((b, s, 1), F32),
            jax.ShapeDtypeStruct((b, 1, 1), F32),
        ],
    )(x, processed, prior_hidden)
    prior_loss = jnp.sum(dchsum) / (b * s)

    gr = g.reshape(b, 1, s)
    rank = pl.pallas_call(
        _rank_kernel,
        grid=(b, nb),
        in_specs=[
            pl.BlockSpec((1, bs, 1), lambda bi, si: (bi, si, 0)),
            pl.BlockSpec((1, 1, s), lambda bi, si: (bi, 0, 0)),
        ],
        out_specs=pl.BlockSpec((1, bs, 1), lambda bi, si: (bi, si, 0)),
        out_shape=jax.ShapeDtypeStruct((b, s, 1), F32),
    )(g, gr)

    sel, gate = pl.pallas_call(
        functools.partial(_gather_kernel, K=k),
        grid=(b, nb),
        in_specs=[
            pl.BlockSpec((1, bs, 1), lambda bi, si: (bi, si, 0)),
            pl.BlockSpec((1, bs, 1), lambda bi, si: (bi, si, 0)),
            pl.BlockSpec((1, bs, HID), lambda bi, si: (bi, si, 0)),
        ],
        out_specs=[
            pl.BlockSpec((1, k, HID), lambda bi, si: (bi, 0, 0)),
            pl.BlockSpec((1, k, 1), lambda bi, si: (bi, 0, 0)),
        ],
        out_shape=[
            jax.ShapeDtypeStruct((b, k, HID), F32),
            jax.ShapeDtypeStruct((b, k, 1), F32),
        ],
    )(rank, g, processed)

    cos_k, sin_k = _rope_tables(k)
    dyn = _decoder_layer(sel, params, "dyn", cos_k, sin_k)

    fin = pl.pallas_call(
        functools.partial(_scatter_kernel, K=k),
        grid=(b, nb),
        in_specs=[
            pl.BlockSpec((1, bs, 1), lambda bi, si: (bi, si, 0)),
            pl.BlockSpec((1, bs, HID), lambda bi, si: (bi, si, 0)),
            pl.BlockSpec((1, k, HID), lambda bi, si: (bi, 0, 0)),
            pl.BlockSpec((1, k, HID), lambda bi, si: (bi, 0, 0)),
            pl.BlockSpec((1, k, 1), lambda bi, si: (bi, 0, 0)),
        ],
        out_specs=pl.BlockSpec((1, bs, HID), lambda bi, si: (bi, si, 0)),
        out_shape=jax.ShapeDtypeStruct((b, s, HID), F32),
    )(rank, processed, sel, dyn, gate)

    return (fin, prior_loss)


# speedup vs baseline: 1.3897x; 1.0157x over previous
import functools

import jax
import jax.numpy as jnp
from jax.experimental import pallas as pl

HID = 1024
NH = 16
HD = HID // NH
FFN = 2816
PFFN = 512
EPS = 1e-6
BETA_CE = 1.0
BETA_CU = 1.0
CAP = 0.5

F32 = jnp.float32
BF16 = jnp.bfloat16


def _rms(x, w):
    v = jnp.mean(x * x, axis=-1, keepdims=True)
    return x * jax.lax.rsqrt(v + EPS) * w


def _qkv_kernel(x_ref, ln_ref, wq_ref, wk_ref, wv_ref, cos_ref, sin_ref,
                q_ref, k_ref, v_ref):
    h = _rms(x_ref[0], ln_ref[...])
    q = jnp.dot(h, wq_ref[...], preferred_element_type=F32)
    k = jnp.dot(h, wk_ref[...], preferred_element_type=F32)
    v = jnp.dot(h, wv_ref[...], preferred_element_type=F32)
    cos = cos_ref[...]
    sin = sin_ref[...]
    half = HD // 2
    for n in range(NH):
        sl = slice(n * HD, (n + 1) * HD)
        qn = q[:, sl]
        kn = k[:, sl]
        qrot = jnp.concatenate([-qn[:, half:], qn[:, :half]], axis=1)
        krot = jnp.concatenate([-kn[:, half:], kn[:, :half]], axis=1)
        q_ref[0, n] = (qn * cos + qrot * sin).astype(BF16)
        k_ref[0, n] = (kn * cos + krot * sin).astype(BF16)
        v_ref[0, n] = v[:, sl].astype(BF16)


def _attn_kernel(q_ref, k_ref, v_ref, o_ref):
    q = q_ref[0, 0]
    si = pl.program_id(2)
    bq = q.shape[0]
    hd = q.shape[1]

    def body(j, carry):
        l, acc = carry
        k = k_ref[0, 0, pl.ds(j * bq, bq), :]
        v = v_ref[0, 0, pl.ds(j * bq, bq), :]
        s = jax.lax.dot_general(q, k, (((1,), (1,)), ((), ())),
                                preferred_element_type=F32) * 0.125
        p = jnp.exp(s)
        l = l + jnp.sum(p, axis=1, keepdims=True)
        acc = acc + jax.lax.dot_general(p.astype(BF16), v,
                                        (((1,), (0,)), ((), ())),
                                        preferred_element_type=F32)
        return l, acc

    init = (jnp.zeros((bq, 1), F32), jnp.zeros((bq, hd), F32))
    l, acc = jax.lax.fori_loop(0, si, body, init)

    k = k_ref[0, 0, pl.ds(si * bq, bq), :]
    v = v_ref[0, 0, pl.ds(si * bq, bq), :]
    s = jax.lax.dot_general(q, k, (((1,), (1,)), ((), ())),
                            preferred_element_type=F32) * 0.125
    row = jax.lax.broadcasted_iota(jnp.int32, (bq, bq), 0)
    col = jax.lax.broadcasted_iota(jnp.int32, (bq, bq), 1)
    p = jnp.where(col <= row, jnp.exp(s), 0.0)
    l = l + jnp.sum(p, axis=1, keepdims=True)
    acc = acc + jax.lax.dot_general(p.astype(BF16), v,
                                    (((1,), (0,)), ((), ())),
                                    preferred_element_type=F32)
    o_ref[0, 0] = (acc / l).astype(BF16)


def _oproj_kernel(o_ref, wo_ref, x_ref, y_ref):
    oc = jnp.concatenate([o_ref[0, n] for n in range(NH)], axis=1)
    y_ref[0] = x_ref[0] + jnp.dot(oc, wo_ref[...], preferred_element_type=F32)


def _mlp_kernel(x_ref, ln_ref, wg_ref, wu_ref, wd_ref, o_ref):
    x = x_ref[0]
    h = _rms(x, ln_ref[...])
    g = jnp.dot(h, wg_ref[...], preferred_element_type=F32)
    u = jnp.dot(h, wu_ref[...], preferred_element_type=F32)
    a = g * jax.nn.sigmoid(g) * u
    part = jnp.dot(a, wd_ref[...], preferred_element_type=F32)

    @pl.when(pl.program_id(2) == 0)
    def _init():
        o_ref[0] = x + part

    @pl.when(pl.program_id(2) > 0)
    def _acc():
        o_ref[0] += part


def _gate_kernel(x_ref, proc_ref, prior_ref, g_ref, dch_ref):
    x = x_ref[0]
    p = proc_ref[0]
    q = prior_ref[0]
    ar = p - x
    dst = jnp.sum(ar * ar, axis=1, keepdims=True) * (1.0 / HID)
    df = p - q
    dch = jnp.sum(df * df, axis=1, keepdims=True) * (1.0 / HID)
    g_ref[0] = jax.nn.sigmoid(BETA_CE * dst - BETA_CU * dch)
    s = jnp.reshape(jnp.sum(dch), (1, 1, 1))

    @pl.when(pl.program_id(1) == 0)
    def _init():
        dch_ref[...] = s

    @pl.when(pl.program_id(1) > 0)
    def _acc():
        dch_ref[...] += s


def _rank_kernel(gc_ref, gr_ref, rank_ref):
    gc = gc_ref[0]
    gr = gr_ref[0]
    bs = gc.shape[0]
    s = gr.shape[1]
    base = pl.program_id(1) * bs
    i = jax.lax.broadcasted_iota(jnp.int32, (bs, s), 0) + base
    j = jax.lax.broadcasted_iota(jnp.int32, (bs, s), 1)
    gt = (gr > gc).astype(F32)
    eq = ((gr == gc) & (j < i)).astype(F32)
    rank_ref[0] = jnp.sum(gt + eq, axis=1, keepdims=True)


def _gather_kernel(rank_ref, g_ref, proc_ref, sel_ref, gate_ref, *, K):
    r = rank_ref[0].astype(jnp.int32)
    jr = jax.lax.broadcasted_iota(jnp.int32, (r.shape[0], K), 1)
    q = (r == jr).astype(F32)
    selp = jax.lax.dot_general(q, proc_ref[0], (((0,), (0,)), ((), ())),
                               preferred_element_type=F32)
    gatep = jax.lax.dot_general(q, g_ref[0], (((0,), (0,)), ((), ())),
                                preferred_element_type=F32)

    @pl.when(pl.program_id(1) == 0)
    def _init():
        sel_ref[0] = selp
        gate_ref[0] = gatep

    @pl.when(pl.program_id(1) > 0)
    def _acc():
        sel_ref[0] += selp
        gate_ref[0] += gatep


def _scatter_kernel(rank_ref, proc_ref, sel_ref, dyn_ref, gate_ref, fin_ref,
                    *, K):
    r = rank_ref[0].astype(jnp.int32)
    w = (r == jax.lax.broadcasted_iota(jnp.int32,
                                       (r.shape[0], K), 1)).astype(F32)
    sel = sel_ref[0]
    dyn = dyn_ref[0]
    gate = gate_ref[0]
    gated = sel + gate * (dyn - sel)
    msel = (r < K).astype(F32)
    fin_ref[0] = (jnp.dot(w, gated, preferred_element_type=F32)
                  + (1.0 - msel) * proc_ref[0])


def _rope_tables(s):
    pos = jnp.arange(s, dtype=F32)
    inv = 1.0 / (10000.0 ** (jnp.arange(0, HD, 2, dtype=F32) / HD))
    f = pos[:, None] * inv[None, :]
    emb = jnp.concatenate([f, f], axis=-1)
    return jnp.cos(emb), jnp.sin(emb)


def _decoder_layer(x, p, pre, cos, sin):
    b, s, _ = x.shape
    bs = min(256, s)
    bq = min(512, s)
    nb = s // bs
    ln1 = p[pre + "_ln1"].reshape(1, HID)

    q, k, v = pl.pallas_call(
        _qkv_kernel,
        grid=(b, nb),
        in_specs=[
            pl.BlockSpec((1, bs, HID), lambda bi, si: (bi, si, 0)),
            pl.BlockSpec((1, HID), lambda bi, si: (0, 0)),
            pl.BlockSpec((HID, HID), lambda bi, si: (0, 0)),
            pl.BlockSpec((HID, HID), lambda bi, si: (0, 0)),
            pl.BlockSpec((HID, HID), lambda bi, si: (0, 0)),
            pl.BlockSpec((bs, HD), lambda bi, si: (si, 0)),
            pl.BlockSpec((bs, HD), lambda bi, si: (si, 0)),
        ],
        out_specs=[
            pl.BlockSpec((1, NH, bs, HD), lambda bi, si: (bi, 0, si, 0)),
            pl.BlockSpec((1, NH, bs, HD), lambda bi, si: (bi, 0, si, 0)),
            pl.BlockSpec((1, NH, bs, HD), lambda bi, si: (bi, 0, si, 0)),
        ],
        out_shape=[jax.ShapeDtypeStruct((b, NH, s, HD), BF16)] * 3,
    )(x, ln1, p[pre + "_wq"], p[pre + "_wk"], p[pre + "_wv"], cos, sin)

    o = pl.pallas_call(
        _attn_kernel,
        grid=(b, NH, s // bq),
        in_specs=[
            pl.BlockSpec((1, 1, bq, HD), lambda bi, hi, si: (bi, hi, si, 0)),
            pl.BlockSpec((1, 1, s, HD), lambda bi, hi, si: (bi, hi, 0, 0)),
            pl.BlockSpec((1, 1, s, HD), lambda bi, hi, si: (bi, hi, 0, 0)),
        ],
        out_specs=pl.BlockSpec((1, 1, bq, HD),
                               lambda bi, hi, si: (bi, hi, si, 0)),
        out_shape=jax.ShapeDtypeStruct((b, NH, s, HD), BF16),
    )(q, k, v)

    x2 = pl.pallas_call(
        _oproj_kernel,
        grid=(b, nb),
        in_specs=[
            pl.BlockSpec((1, NH, bs, HD), lambda bi, si: (bi, 0, si, 0)),
            pl.BlockSpec((HID, HID), lambda bi, si: (0, 0)),
            pl.BlockSpec((1, bs, HID), lambda bi, si: (bi, si, 0)),
        ],
        out_specs=pl.BlockSpec((1, bs, HID), lambda bi, si: (bi, si, 0)),
        out_shape=jax.ShapeDtypeStruct(x.shape, F32),
    )(o, p[pre + "_wo"].astype(BF16), x)

    return _mlp(x2, p[pre + "_ln2"], p[pre + "_wg"], p[pre + "_wu"],
                p[pre + "_wd"])


def _mlp(x, ln, wg, wu, wd, fb=None):
    b, s, _ = x.shape
    bs = min(256, s)
    nb = s // bs
    f = wg.shape[1]
    if fb is None:
        fb = f
    nf = f // fb
    return pl.pallas_call(
        _mlp_kernel,
        grid=(b, nb, nf),
        in_specs=[
            pl.BlockSpec((1, bs, HID), lambda bi, si, fi: (bi, si, 0)),
            pl.BlockSpec((1, HID), lambda bi, si, fi: (0, 0)),
            pl.BlockSpec((HID, fb), lambda bi, si, fi: (0, fi)),
            pl.BlockSpec((HID, fb), lambda bi, si, fi: (0, fi)),
            pl.BlockSpec((fb, HID), lambda bi, si, fi: (fi, 0)),
        ],
        out_specs=pl.BlockSpec((1, bs, HID), lambda bi, si, fi: (bi, si, 0)),
        out_shape=jax.ShapeDtypeStruct(x.shape, F32),
    )(x, ln.reshape(1, HID), wg, wu, wd)


def kernel(hidden_states, params):
    x = hidden_states
    b, s, _ = x.shape
    k = max(1, int(s * CAP))
    bs = min(256, s)
    nb = s // bs

    cos_s, sin_s = _rope_tables(s)
    processed = _decoder_layer(x, params, "dec", cos_s, sin_s)
    prior_hidden = _mlp(x, params["prior_ln"], params["prior_wg"],
                        params["prior_wu"], params["prior_wd"])

    g, dchsum = pl.pallas_call(
        _gate_kernel,
        grid=(b, nb),
        in_specs=[
            pl.BlockSpec((1, bs, HID), lambda bi, si: (bi, si, 0)),
            pl.BlockSpec((1, bs, HID), lambda bi, si: (bi, si, 0)),
            pl.BlockSpec((1, bs, HID), lambda bi, si: (bi, si, 0)),
        ],
        out_specs=[
            pl.BlockSpec((1, bs, 1), lambda bi, si: (bi, si, 0)),
            pl.BlockSpec((1, 1, 1), lambda bi, si: (bi, 0, 0)),
        ],
        out_shape=[
            jax.ShapeDtypeStruct((b, s, 1), F32),
            jax.ShapeDtypeStruct((b, 1, 1), F32),
        ],
    )(x, processed, prior_hidden)
    prior_loss = jnp.sum(dchsum) / (b * s)

    gr = g.reshape(b, 1, s)
    rank = pl.pallas_call(
        _rank_kernel,
        grid=(b, nb),
        in_specs=[
            pl.BlockSpec((1, bs, 1), lambda bi, si: (bi, si, 0)),
            pl.BlockSpec((1, 1, s), lambda bi, si: (bi, 0, 0)),
        ],
        out_specs=pl.BlockSpec((1, bs, 1), lambda bi, si: (bi, si, 0)),
        out_shape=jax.ShapeDtypeStruct((b, s, 1), F32),
    )(g, gr)

    sel, gate = pl.pallas_call(
        functools.partial(_gather_kernel, K=k),
        grid=(b, nb),
        in_specs=[
            pl.BlockSpec((1, bs, 1), lambda bi, si: (bi, si, 0)),
            pl.BlockSpec((1, bs, 1), lambda bi, si: (bi, si, 0)),
            pl.BlockSpec((1, bs, HID), lambda bi, si: (bi, si, 0)),
        ],
        out_specs=[
            pl.BlockSpec((1, k, HID), lambda bi, si: (bi, 0, 0)),
            pl.BlockSpec((1, k, 1), lambda bi, si: (bi, 0, 0)),
        ],
        out_shape=[
            jax.ShapeDtypeStruct((b, k, HID), F32),
            jax.ShapeDtypeStruct((b, k, 1), F32),
        ],
    )(rank, g, processed)

    cos_k, sin_k = _rope_tables(k)
    dyn = _decoder_layer(sel, params, "dyn", cos_k, sin_k)

    fin = pl.pallas_call(
        functools.partial(_scatter_kernel, K=k),
        grid=(b, nb),
        in_specs=[
            pl.BlockSpec((1, bs, 1), lambda bi, si: (bi, si, 0)),
            pl.BlockSpec((1, bs, HID), lambda bi, si: (bi, si, 0)),
            pl.BlockSpec((1, k, HID), lambda bi, si: (bi, 0, 0)),
            pl.BlockSpec((1, k, HID), lambda bi, si: (bi, 0, 0)),
            pl.BlockSpec((1, k, 1), lambda bi, si: (bi, 0, 0)),
        ],
        out_specs=pl.BlockSpec((1, bs, HID), lambda bi, si: (bi, si, 0)),
        out_shape=jax.ShapeDtypeStruct((b, s, HID), F32),
    )(rank, processed, sel, dyn, gate)

    return (fin, prior_loss)


# speedup vs baseline: 1.4025x; 1.0093x over previous
import functools

import jax
import jax.numpy as jnp
from jax.experimental import pallas as pl

HID = 1024
NH = 16
HD = HID // NH
FFN = 2816
PFFN = 512
EPS = 1e-6
BETA_CE = 1.0
BETA_CU = 1.0
CAP = 0.5

F32 = jnp.float32
BF16 = jnp.bfloat16


def _rms(x, w):
    v = jnp.mean(x * x, axis=-1, keepdims=True)
    return x * jax.lax.rsqrt(v + EPS) * w


def _qkv_kernel(x_ref, ln_ref, wq_ref, wk_ref, wv_ref, cos_ref, sin_ref,
                q_ref, k_ref, v_ref):
    h = _rms(x_ref[0], ln_ref[...])
    q = jnp.dot(h, wq_ref[...], preferred_element_type=F32)
    k = jnp.dot(h, wk_ref[...], preferred_element_type=F32)
    v = jnp.dot(h, wv_ref[...], preferred_element_type=F32)
    cos = cos_ref[...]
    sin = sin_ref[...]
    half = HD // 2
    for n in range(NH):
        sl = slice(n * HD, (n + 1) * HD)
        qn = q[:, sl]
        kn = k[:, sl]
        qrot = jnp.concatenate([-qn[:, half:], qn[:, :half]], axis=1)
        krot = jnp.concatenate([-kn[:, half:], kn[:, :half]], axis=1)
        q_ref[0, n] = (qn * cos + qrot * sin).astype(BF16)
        k_ref[0, n] = (kn * cos + krot * sin).astype(BF16)
        v_ref[0, n] = v[:, sl].astype(BF16)


def _attn_kernel(q_ref, k_ref, v_ref, m_ref, o_ref):
    q = q_ref[0, 0]
    si = pl.program_id(2)
    bq = q.shape[0]
    ones = jnp.ones((bq, 1), BF16)

    def chunk(off, acc, masked):
        k = k_ref[0, 0, pl.ds(off, bq), :]
        v = v_ref[0, 0, pl.ds(off, bq), :]
        s = jax.lax.dot_general(q, k, (((1,), (1,)), ((), ())),
                                preferred_element_type=F32) * 0.125
        p = jnp.exp(s)
        if masked:
            p = p * m_ref[...]
        va = jnp.concatenate([v, ones], axis=1)
        return acc + jax.lax.dot_general(p.astype(BF16), va,
                                         (((1,), (0,)), ((), ())),
                                         preferred_element_type=F32)

    def body(j, acc):
        return chunk(j * bq, acc, False)

    acc = jax.lax.fori_loop(0, si, body, jnp.zeros((bq, HD + 1), F32))
    acc = chunk(si * bq, acc, True)
    o_ref[0, 0] = (acc[:, :HD] / acc[:, HD:HD + 1]).astype(BF16)


def _oproj_kernel(o_ref, wo_ref, x_ref, y_ref):
    oc = jnp.concatenate([o_ref[0, n] for n in range(NH)], axis=1)
    y_ref[0] = x_ref[0] + jnp.dot(oc, wo_ref[...], preferred_element_type=F32)


def _mlp_kernel(x_ref, ln_ref, wg_ref, wu_ref, wd_ref, o_ref):
    x = x_ref[0]
    h = _rms(x, ln_ref[...])
    g = jnp.dot(h, wg_ref[...], preferred_element_type=F32)
    u = jnp.dot(h, wu_ref[...], preferred_element_type=F32)
    a = g * jax.nn.sigmoid(g) * u
    part = jnp.dot(a, wd_ref[...], preferred_element_type=F32)

    @pl.when(pl.program_id(2) == 0)
    def _init():
        o_ref[0] = x + part

    @pl.when(pl.program_id(2) > 0)
    def _acc():
        o_ref[0] += part


def _gate_kernel(x_ref, proc_ref, prior_ref, g_ref, dch_ref):
    x = x_ref[0]
    p = proc_ref[0]
    q = prior_ref[0]
    ar = p - x
    dst = jnp.sum(ar * ar, axis=1, keepdims=True) * (1.0 / HID)
    df = p - q
    dch = jnp.sum(df * df, axis=1, keepdims=True) * (1.0 / HID)
    g_ref[0] = jax.nn.sigmoid(BETA_CE * dst - BETA_CU * dch)
    s = jnp.reshape(jnp.sum(dch), (1, 1, 1))

    @pl.when(pl.program_id(1) == 0)
    def _init():
        dch_ref[...] = s

    @pl.when(pl.program_id(1) > 0)
    def _acc():
        dch_ref[...] += s


def _rank_kernel(gc_ref, gr_ref, rank_ref):
    gc = gc_ref[0]
    gr = gr_ref[0]
    bs = gc.shape[0]
    s = gr.shape[1]
    base = pl.program_id(1) * bs
    i = jax.lax.broadcasted_iota(jnp.int32, (bs, s), 0) + base
    j = jax.lax.broadcasted_iota(jnp.int32, (bs, s), 1)
    gt = (gr > gc).astype(F32)
    eq = ((gr == gc) & (j < i)).astype(F32)
    rank_ref[0] = jnp.sum(gt + eq, axis=1, keepdims=True)


def _gather_kernel(rank_ref, g_ref, proc_ref, sel_ref, gate_ref, *, K):
    r = rank_ref[0].astype(jnp.int32)
    jr = jax.lax.broadcasted_iota(jnp.int32, (r.shape[0], K), 1)
    q = (r == jr).astype(F32)
    selp = jax.lax.dot_general(q, proc_ref[0], (((0,), (0,)), ((), ())),
                               preferred_element_type=F32)
    gatep = jax.lax.dot_general(q, g_ref[0], (((0,), (0,)), ((), ())),
                                preferred_element_type=F32)

    @pl.when(pl.program_id(1) == 0)
    def _init():
        sel_ref[0] = selp
        gate_ref[0] = gatep

    @pl.when(pl.program_id(1) > 0)
    def _acc():
        sel_ref[0] += selp
        gate_ref[0] += gatep


def _scatter_kernel(rank_ref, proc_ref, sel_ref, dyn_ref, gate_ref, fin_ref,
                    *, K):
    r = rank_ref[0].astype(jnp.int32)
    w = (r == jax.lax.broadcasted_iota(jnp.int32,
                                       (r.shape[0], K), 1)).astype(F32)
    sel = sel_ref[0]
    dyn = dyn_ref[0]
    gate = gate_ref[0]
    gated = sel + gate * (dyn - sel)
    msel = (r < K).astype(F32)
    fin_ref[0] = (jnp.dot(w, gated, preferred_element_type=F32)
                  + (1.0 - msel) * proc_ref[0])


def _rope_tables(s):
    pos = jnp.arange(s, dtype=F32)
    inv = 1.0 / (10000.0 ** (jnp.arange(0, HD, 2, dtype=F32) / HD))
    f = pos[:, None] * inv[None, :]
    emb = jnp.concatenate([f, f], axis=-1)
    return jnp.cos(emb), jnp.sin(emb)


def _decoder_layer(x, p, pre, cos, sin):
    b, s, _ = x.shape
    bs = min(256, s)
    bq = min(512, s)
    nb = s // bs
    ln1 = p[pre + "_ln1"].reshape(1, HID)

    q, k, v = pl.pallas_call(
        _qkv_kernel,
        grid=(b, nb),
        in_specs=[
            pl.BlockSpec((1, bs, HID), lambda bi, si: (bi, si, 0)),
            pl.BlockSpec((1, HID), lambda bi, si: (0, 0)),
            pl.BlockSpec((HID, HID), lambda bi, si: (0, 0)),
            pl.BlockSpec((HID, HID), lambda bi, si: (0, 0)),
            pl.BlockSpec((HID, HID), lambda bi, si: (0, 0)),
            pl.BlockSpec((bs, HD), lambda bi, si: (si, 0)),
            pl.BlockSpec((bs, HD), lambda bi, si: (si, 0)),
        ],
        out_specs=[
            pl.BlockSpec((1, NH, bs, HD), lambda bi, si: (bi, 0, si, 0)),
            pl.BlockSpec((1, NH, bs, HD), lambda bi, si: (bi, 0, si, 0)),
            pl.BlockSpec((1, NH, bs, HD), lambda bi, si: (bi, 0, si, 0)),
        ],
        out_shape=[jax.ShapeDtypeStruct((b, NH, s, HD), BF16)] * 3,
    )(x, ln1, p[pre + "_wq"], p[pre + "_wk"], p[pre + "_wv"], cos, sin)

    tri = jnp.tril(jnp.ones((bq, bq), F32))
    o = pl.pallas_call(
        _attn_kernel,
        grid=(b, NH, s // bq),
        in_specs=[
            pl.BlockSpec((1, 1, bq, HD), lambda bi, hi, si: (bi, hi, si, 0)),
            pl.BlockSpec((1, 1, s, HD), lambda bi, hi, si: (bi, hi, 0, 0)),
            pl.BlockSpec((1, 1, s, HD), lambda bi, hi, si: (bi, hi, 0, 0)),
            pl.BlockSpec((bq, bq), lambda bi, hi, si: (0, 0)),
        ],
        out_specs=pl.BlockSpec((1, 1, bq, HD),
                               lambda bi, hi, si: (bi, hi, si, 0)),
        out_shape=jax.ShapeDtypeStruct((b, NH, s, HD), BF16),
    )(q, k, v, tri)

    x2 = pl.pallas_call(
        _oproj_kernel,
        grid=(b, nb),
        in_specs=[
            pl.BlockSpec((1, NH, bs, HD), lambda bi, si: (bi, 0, si, 0)),
            pl.BlockSpec((HID, HID), lambda bi, si: (0, 0)),
            pl.BlockSpec((1, bs, HID), lambda bi, si: (bi, si, 0)),
        ],
        out_specs=pl.BlockSpec((1, bs, HID), lambda bi, si: (bi, si, 0)),
        out_shape=jax.ShapeDtypeStruct(x.shape, F32),
    )(o, p[pre + "_wo"].astype(BF16), x)

    return _mlp(x2, p[pre + "_ln2"], p[pre + "_wg"], p[pre + "_wu"],
                p[pre + "_wd"])


def _mlp(x, ln, wg, wu, wd, fb=None):
    b, s, _ = x.shape
    bs = min(256, s)
    nb = s // bs
    f = wg.shape[1]
    if fb is None:
        fb = f
    nf = f // fb
    return pl.pallas_call(
        _mlp_kernel,
        grid=(b, nb, nf),
        in_specs=[
            pl.BlockSpec((1, bs, HID), lambda bi, si, fi: (bi, si, 0)),
            pl.BlockSpec((1, HID), lambda bi, si, fi: (0, 0)),
            pl.BlockSpec((HID, fb), lambda bi, si, fi: (0, fi)),
            pl.BlockSpec((HID, fb), lambda bi, si, fi: (0, fi)),
            pl.BlockSpec((fb, HID), lambda bi, si, fi: (fi, 0)),
        ],
        out_specs=pl.BlockSpec((1, bs, HID), lambda bi, si, fi: (bi, si, 0)),
        out_shape=jax.ShapeDtypeStruct(x.shape, F32),
    )(x, ln.reshape(1, HID), wg, wu, wd)


def kernel(hidden_states, params):
    x = hidden_states
    b, s, _ = x.shape
    k = max(1, int(s * CAP))
    bs = min(256, s)
    nb = s // bs

    cos_s, sin_s = _rope_tables(s)
    processed = _decoder_layer(x, params, "dec", cos_s, sin_s)
    prior_hidden = _mlp(x, params["prior_ln"], params["prior_wg"],
                        params["prior_wu"], params["prior_wd"])

    g, dchsum = pl.pallas_call(
        _gate_kernel,
        grid=(b, nb),
        in_specs=[
            pl.BlockSpec((1, bs, HID), lambda bi, si: (bi, si, 0)),
            pl.BlockSpec((1, bs, HID), lambda bi, si: (bi, si, 0)),
            pl.BlockSpec((1, bs, HID), lambda bi, si: (bi, si, 0)),
        ],
        out_specs=[
            pl.BlockSpec((1, bs, 1), lambda bi, si: (bi, si, 0)),
            pl.BlockSpec((1, 1, 1), lambda bi, si: (bi, 0, 0)),
        ],
        out_shape=[
            jax.ShapeDtypeStruct((b, s, 1), F32),
            jax.ShapeDtypeStruct((b, 1, 1), F32),
        ],
    )(x, processed, prior_hidden)
    prior_loss = jnp.sum(dchsum) / (b * s)

    gr = g.reshape(b, 1, s)
    rank = pl.pallas_call(
        _rank_kernel,
        grid=(b, nb),
        in_specs=[
            pl.BlockSpec((1, bs, 1), lambda bi, si: (bi, si, 0)),
            pl.BlockSpec((1, 1, s), lambda bi, si: (bi, 0, 0)),
        ],
        out_specs=pl.BlockSpec((1, bs, 1), lambda bi, si: (bi, si, 0)),
        out_shape=jax.ShapeDtypeStruct((b, s, 1), F32),
    )(g, gr)

    sel, gate = pl.pallas_call(
        functools.partial(_gather_kernel, K=k),
        grid=(b, nb),
        in_specs=[
            pl.BlockSpec((1, bs, 1), lambda bi, si: (bi, si, 0)),
            pl.BlockSpec((1, bs, 1), lambda bi, si: (bi, si, 0)),
            pl.BlockSpec((1, bs, HID), lambda bi, si: (bi, si, 0)),
        ],
        out_specs=[
            pl.BlockSpec((1, k, HID), lambda bi, si: (bi, 0, 0)),
            pl.BlockSpec((1, k, 1), lambda bi, si: (bi, 0, 0)),
        ],
        out_shape=[
            jax.ShapeDtypeStruct((b, k, HID), F32),
            jax.ShapeDtypeStruct((b, k, 1), F32),
        ],
    )(rank, g, processed)

    cos_k, sin_k = _rope_tables(k)
    dyn = _decoder_layer(sel, params, "dyn", cos_k, sin_k)

    fin = pl.pallas_call(
        functools.partial(_scatter_kernel, K=k),
        grid=(b, nb),
        in_specs=[
            pl.BlockSpec((1, bs, 1), lambda bi, si: (bi, si, 0)),
            pl.BlockSpec((1, bs, HID), lambda bi, si: (bi, si, 0)),
            pl.BlockSpec((1, k, HID), lambda bi, si: (bi, 0, 0)),
            pl.BlockSpec((1, k, HID), lambda bi, si: (bi, 0, 0)),
            pl.BlockSpec((1, k, 1), lambda bi, si: (bi, 0, 0)),
        ],
        out_specs=pl.BlockSpec((1, bs, HID), lambda bi, si: (bi, si, 0)),
        out_shape=jax.ShapeDtypeStruct((b, s, HID), F32),
    )(rank, processed, sel, dyn, gate)

    return (fin, prior_loss)
